# baseline (speedup 1.0000x reference)
import functools

import jax
import jax.numpy as jnp
import numpy as np
from jax import lax
from jax.experimental import pallas as pl
from jax.experimental.pallas import tpu as pltpu
from jax.experimental.pallas import tpu_sc as plsc

_NC, _NS, _LANES = 2, 16, 16
_NW = _NC * _NS
_ROWS = 512
_COLS = 4096
_BLK = 64
_NB = _COLS // _BLK
_NG = _NB // _LANES
_RPW = _ROWS // _NW


def _eytzinger_perm(n=15, depth=4):
    p = np.zeros(n, dtype=np.int32)

    def build(node, lo, hi):
        if lo >= hi:
            return
        mid = (lo + hi) // 2
        p[node - 1] = mid
        build(2 * node, lo, mid)
        build(2 * node + 1, mid + 1, hi)

    build(1, 0, n)
    return p


_EY_PERM = _eytzinger_perm()


def _sc_body(x_hbm, ey_hbm, u_hbm,
             rec_hbm, q_hbm, sq_hbm, smn_hbm, smx_hbm, xmn_hbm,
             xrow, rrow, qrow, eyv, uv, scb, sdb, xmb, sqb, smb, sxb):
    wid = lax.axis_index("c") * _NS + lax.axis_index("s")
    base_row = wid * _RPW
    pltpu.sync_copy(ey_hbm, eyv)
    pltpu.sync_copy(u_hbm, uv)
    plsc.subcore_barrier()
    lane = lax.iota(jnp.int32, _LANES)
    lane64 = lane * _BLK

    def row_step(r, carry):
        roff = (base_row + r) * _COLS
        pltpu.sync_copy(x_hbm.at[pl.ds(roff, _COLS)], xrow)

        for g in range(_NG):
            gbase = lane64 + g * (_LANES * _BLK)

            def mm_step(e, c, gbase=gbase):
                vmn, vmx = c
                xe = plsc.load_gather(xrow, [gbase + e])
                return jnp.minimum(vmn, xe), jnp.maximum(vmx, xe)

            big = jnp.full((_LANES,), jnp.inf, jnp.float32)
            vmn, vmx = lax.fori_loop(0, _BLK, mm_step, (big, -big), unroll=8)
            xmb[pl.ds(r * _NB + g * _LANES, _LANES)] = vmn
            scb[pl.ds(g * _LANES, _LANES)] = vmx - vmn

        s0 = scb[pl.ds(0, _LANES)]
        s1 = scb[pl.ds(_LANES, _LANES)]
        s2 = scb[pl.ds(2 * _LANES, _LANES)]
        s3 = scb[pl.ds(3 * _LANES, _LANES)]
        smin = jnp.min(jnp.minimum(jnp.minimum(s0, s1), jnp.minimum(s2, s3)))
        smax = jnp.max(jnp.maximum(jnp.maximum(s0, s1), jnp.maximum(s2, s3)))
        dd = smax - smin + jnp.float32(1e-8)
        rng = smax - smin
        for g in range(_NG):
            sc = scb[pl.ds(g * _LANES, _LANES)]
            v = (sc - smin) / dd * jnp.float32(255.0)
            qi = (v + jnp.float32(0.5)).astype(jnp.int32)
            sqb[pl.ds(r * _NB + g * _LANES, _LANES)] = qi
            sdb[pl.ds(g * _LANES, _LANES)] = (
                smin + qi.astype(jnp.float32) / jnp.float32(255.0) * rng)
        msk = lane == r
        plsc.store_scatter(smb, [lane], jnp.full((_LANES,), smin, jnp.float32),
                           mask=msk)
        plsc.store_scatter(sxb, [lane], jnp.full((_LANES,), smax, jnp.float32),
                           mask=msk)

        for g in range(_NG):
            gbase = lane64 + g * (_LANES * _BLK)
            vmn = xmb[pl.ds(r * _NB + g * _LANES, _LANES)]
            sc = scb[pl.ds(g * _LANES, _LANES)]
            sd = sdb[pl.ds(g * _LANES, _LANES)]
            inv2 = jnp.float32(2.0) / (sc + jnp.float32(1e-8))

            def q_step(e, c, gbase=gbase, vmn=vmn, sd=sd, inv2=inv2):
                idx = gbase + e
                xe = plsc.load_gather(xrow, [idx])
                xn = (xe - vmn) * inv2 - jnp.float32(1.0)
                k = jnp.where(xe == xe, 1, 2).astype(jnp.int32)
                for _ in range(4):
                    hk = plsc.load_gather(eyv, [k - 1])
                    k = 2 * k + jnp.where(xn > hk, 1, 0).astype(jnp.int32)
                qv = k - 16
                uu = plsc.load_gather(uv, [qv])
                plsc.store_scatter(rrow, [idx], uu * sd + vmn)
                plsc.store_scatter(qrow, [idx], qv)
                return c

            lax.fori_loop(0, _BLK, q_step, 0, unroll=8)

        pltpu.sync_copy(rrow, rec_hbm.at[pl.ds(roff, _COLS)])
        pltpu.sync_copy(qrow, q_hbm.at[pl.ds(roff, _COLS)])
        return carry

    lax.fori_loop(0, _RPW, row_step, 0)
    pltpu.sync_copy(xmb, xmn_hbm.at[pl.ds(base_row * _NB, _RPW * _NB)])
    pltpu.sync_copy(sqb, sq_hbm.at[pl.ds(base_row * _NB, _RPW * _NB)])
    pltpu.sync_copy(smb, smn_hbm.at[pl.ds(base_row, _RPW)])
    pltpu.sync_copy(sxb, smx_hbm.at[pl.ds(base_row, _RPW)])


@functools.cache
def _make_sc_call():
    mesh = plsc.VectorSubcoreMesh(core_axis_name="c", subcore_axis_name="s",
                                  num_cores=_NC, num_subcores=_NS)
    n = _ROWS * _COLS
    return pl.kernel(
        _sc_body,
        out_type=(
            jax.ShapeDtypeStruct((n,), jnp.float32),
            jax.ShapeDtypeStruct((n,), jnp.int32),
            jax.ShapeDtypeStruct((_ROWS * _NB,), jnp.int32),
            jax.ShapeDtypeStruct((_ROWS,), jnp.float32),
            jax.ShapeDtypeStruct((_ROWS,), jnp.float32),
            jax.ShapeDtypeStruct((_ROWS * _NB,), jnp.float32),
        ),
        mesh=mesh,
        compiler_params=pltpu.CompilerParams(needs_layout_passes=False),
        scratch_types=(
            pltpu.VMEM((_COLS,), jnp.float32),
            pltpu.VMEM((_COLS,), jnp.float32),
            pltpu.VMEM((_COLS,), jnp.int32),
            pltpu.VMEM((_LANES,), jnp.float32),
            pltpu.VMEM((_LANES,), jnp.float32),
            pltpu.VMEM((_NB,), jnp.float32),
            pltpu.VMEM((_NB,), jnp.float32),
            pltpu.VMEM((_RPW * _NB,), jnp.float32),
            pltpu.VMEM((_RPW * _NB,), jnp.int32),
            pltpu.VMEM((_RPW,), jnp.float32),
            pltpu.VMEM((_RPW,), jnp.float32),
        ),
    )


def _nf_tables():
    n_levels = 16
    p = (jnp.arange(n_levels, dtype=jnp.float32) + 0.5) / n_levels
    t = jnp.sqrt(2.0) * jax.scipy.special.erfinv(2.0 * p - 1.0)
    t = (t / jnp.max(jnp.abs(t))).astype(jnp.float32)
    u = (t + 1.0) / 2.0
    mids = (t[:-1] + t[1:]) * 0.5
    ey = jnp.concatenate([mids[_EY_PERM], jnp.zeros((1,), jnp.float32)])
    return ey.astype(jnp.float32), u.astype(jnp.float32)


def kernel(x):
    orig_shape = x.shape
    ey16, u16 = _nf_tables()
    xf = x.reshape(-1)
    rec, qf, sq, smn, smx, xmn = _make_sc_call()(xf, ey16, u16)
    return (rec.reshape(orig_shape),
            qf.reshape(_ROWS, _NB, _BLK),
            sq.astype(jnp.uint8).reshape(_ROWS, _NB),
            smn.reshape(_ROWS, 1),
            smx.reshape(_ROWS, 1),
            xmn.reshape(_ROWS, _NB, 1))


# speedup vs baseline: 1.2947x; 1.2947x over previous
import functools

import jax
import jax.numpy as jnp
import numpy as np
from jax import lax
from jax.experimental import pallas as pl
from jax.experimental.pallas import tpu as pltpu
from jax.experimental.pallas import tpu_sc as plsc

_NC, _NS, _LANES = 2, 16, 16
_NW = _NC * _NS
_ROWS = 512
_COLS = 4096
_BLK = 64
_NB = _COLS // _BLK
_NG = _NB // _LANES
_RPW = _ROWS // _NW


def _eytzinger_perm(n=15, depth=4):
    p = np.zeros(n, dtype=np.int32)

    def build(node, lo, hi):
        if lo >= hi:
            return
        mid = (lo + hi) // 2
        p[node - 1] = mid
        build(2 * node, lo, mid)
        build(2 * node + 1, mid + 1, hi)

    build(1, 0, n)
    return p


_EY_PERM = _eytzinger_perm()

_M = (-0.853784441947937, -0.6248890161514282, -0.4795140027999878,
      -0.3638618588447571, -0.2634255886077881, -0.1716436743736267,
      -0.08471819758415222, 0.0, 0.08471819758415222, 0.1716436743736267,
      0.2634255886077881, 0.3638618588447571, 0.4795140027999878,
      0.6248890161514282, 0.853784441947937)


def _sc_body(x_hbm, ey_hbm, u_hbm,
             rec_hbm, q_hbm, sq_hbm, smn_hbm, smx_hbm, xmn_hbm,
             xrow, rrow, qrow, eyv, uv, scb, sdb, xmb, sqb, smb, sxb):
    wid = lax.axis_index("c") * _NS + lax.axis_index("s")
    base_row = wid * _RPW
    pltpu.sync_copy(ey_hbm, eyv)
    pltpu.sync_copy(u_hbm, uv)
    plsc.subcore_barrier()
    lane = lax.iota(jnp.int32, _LANES)
    lane64 = lane * _BLK

    def row_step(r, carry):
        roff = (base_row + r) * _COLS
        pltpu.sync_copy(x_hbm.at[pl.ds(roff, _COLS)], xrow)

        for g in range(_NG):
            gbase = lane64 + g * (_LANES * _BLK)

            def mm_step(e, c, gbase=gbase):
                vmn, vmx = c
                xe = plsc.load_gather(xrow, [gbase + e])
                return jnp.minimum(vmn, xe), jnp.maximum(vmx, xe)

            big = jnp.full((_LANES,), jnp.inf, jnp.float32)
            vmn, vmx = lax.fori_loop(0, _BLK, mm_step, (big, -big), unroll=8)
            xmb[pl.ds(r * _NB + g * _LANES, _LANES)] = vmn
            scb[pl.ds(g * _LANES, _LANES)] = vmx - vmn

        s0 = scb[pl.ds(0, _LANES)]
        s1 = scb[pl.ds(_LANES, _LANES)]
        s2 = scb[pl.ds(2 * _LANES, _LANES)]
        s3 = scb[pl.ds(3 * _LANES, _LANES)]
        smin = jnp.min(jnp.minimum(jnp.minimum(s0, s1), jnp.minimum(s2, s3)))
        smax = jnp.max(jnp.maximum(jnp.maximum(s0, s1), jnp.maximum(s2, s3)))
        dd = smax - smin + jnp.float32(1e-8)
        rng = smax - smin
        for g in range(_NG):
            sc = scb[pl.ds(g * _LANES, _LANES)]
            v = (sc - smin) / dd * jnp.float32(255.0)
            qi = (v + jnp.float32(0.5)).astype(jnp.int32)
            sqb[pl.ds(r * _NB + g * _LANES, _LANES)] = qi
            sdb[pl.ds(g * _LANES, _LANES)] = (
                smin + qi.astype(jnp.float32) / jnp.float32(255.0) * rng)
        msk = lane == r
        plsc.store_scatter(smb, [lane], jnp.full((_LANES,), smin, jnp.float32),
                           mask=msk)
        plsc.store_scatter(sxb, [lane], jnp.full((_LANES,), smax, jnp.float32),
                           mask=msk)

        for g in range(_NG):
            gbase = lane64 + g * (_LANES * _BLK)
            vmn = xmb[pl.ds(r * _NB + g * _LANES, _LANES)]
            sc = scb[pl.ds(g * _LANES, _LANES)]
            sd = sdb[pl.ds(g * _LANES, _LANES)]
            inv2 = jnp.float32(2.0) / (sc + jnp.float32(1e-8))
            xb = -vmn * inv2 - jnp.float32(1.0)
            M = _M

            def q_step(e, c, gbase=gbase, vmn=vmn, sd=sd, inv2=inv2, xb=xb):
                idx = gbase + e
                xe = plsc.load_gather(xrow, [idx])
                xn = xe * inv2 + xb
                b1 = xn > M[7]
                h2 = jnp.where(b1, M[11], M[3])
                b2 = xn > h2
                h3 = jnp.where(b1, jnp.where(b2, M[13], M[9]),
                               jnp.where(b2, M[5], M[1]))
                b3 = xn > h3
                t00 = jnp.where(b3, M[2], M[0])
                t01 = jnp.where(b3, M[6], M[4])
                t10 = jnp.where(b3, M[10], M[8])
                t11 = jnp.where(b3, M[14], M[12])
                h4 = jnp.where(b1, jnp.where(b2, t11, t10),
                               jnp.where(b2, t01, t00))
                b4 = xn > h4
                qv = (jnp.where(b1, 8, 0) + jnp.where(b2, 4, 0)
                      + jnp.where(b3, 2, 0) + jnp.where(b4, 1, 0))
                uu = plsc.load_gather(uv, [qv])
                plsc.store_scatter(rrow, [idx], uu * sd + vmn)
                plsc.store_scatter(qrow, [idx], qv)
                return c

            lax.fori_loop(0, _BLK, q_step, 0, unroll=8)

        pltpu.sync_copy(rrow, rec_hbm.at[pl.ds(roff, _COLS)])
        pltpu.sync_copy(qrow, q_hbm.at[pl.ds(roff, _COLS)])
        return carry

    lax.fori_loop(0, _RPW, row_step, 0)
    pltpu.sync_copy(xmb, xmn_hbm.at[pl.ds(base_row * _NB, _RPW * _NB)])
    pltpu.sync_copy(sqb, sq_hbm.at[pl.ds(base_row * _NB, _RPW * _NB)])
    pltpu.sync_copy(smb, smn_hbm.at[pl.ds(base_row, _RPW)])
    pltpu.sync_copy(sxb, smx_hbm.at[pl.ds(base_row, _RPW)])


@functools.cache
def _make_sc_call():
    mesh = plsc.VectorSubcoreMesh(core_axis_name="c", subcore_axis_name="s",
                                  num_cores=_NC, num_subcores=_NS)
    n = _ROWS * _COLS
    return pl.kernel(
        _sc_body,
        out_type=(
            jax.ShapeDtypeStruct((n,), jnp.float32),
            jax.ShapeDtypeStruct((n,), jnp.int32),
            jax.ShapeDtypeStruct((_ROWS * _NB,), jnp.int32),
            jax.ShapeDtypeStruct((_ROWS,), jnp.float32),
            jax.ShapeDtypeStruct((_ROWS,), jnp.float32),
            jax.ShapeDtypeStruct((_ROWS * _NB,), jnp.float32),
        ),
        mesh=mesh,
        compiler_params=pltpu.CompilerParams(needs_layout_passes=False),
        scratch_types=(
            pltpu.VMEM((_COLS,), jnp.float32),
            pltpu.VMEM((_COLS,), jnp.float32),
            pltpu.VMEM((_COLS,), jnp.int32),
            pltpu.VMEM((_LANES,), jnp.float32),
            pltpu.VMEM((_LANES,), jnp.float32),
            pltpu.VMEM((_NB,), jnp.float32),
            pltpu.VMEM((_NB,), jnp.float32),
            pltpu.VMEM((_RPW * _NB,), jnp.float32),
            pltpu.VMEM((_RPW * _NB,), jnp.int32),
            pltpu.VMEM((_RPW,), jnp.float32),
            pltpu.VMEM((_RPW,), jnp.float32),
        ),
    )


def _nf_tables():
    n_levels = 16
    p = (jnp.arange(n_levels, dtype=jnp.float32) + 0.5) / n_levels
    t = jnp.sqrt(2.0) * jax.scipy.special.erfinv(2.0 * p - 1.0)
    t = (t / jnp.max(jnp.abs(t))).astype(jnp.float32)
    u = (t + 1.0) / 2.0
    mids = (t[:-1] + t[1:]) * 0.5
    ey = jnp.concatenate([mids[_EY_PERM], jnp.zeros((1,), jnp.float32)])
    return ey.astype(jnp.float32), u.astype(jnp.float32)


def kernel(x):
    orig_shape = x.shape
    ey16, u16 = _nf_tables()
    xf = x.reshape(-1)
    rec, qf, sq, smn, smx, xmn = _make_sc_call()(xf, ey16, u16)
    return (rec.reshape(orig_shape),
            qf.reshape(_ROWS, _NB, _BLK),
            sq.astype(jnp.uint8).reshape(_ROWS, _NB),
            smn.reshape(_ROWS, 1),
            smx.reshape(_ROWS, 1),
            xmn.reshape(_ROWS, _NB, 1))


# speedup vs baseline: 2.2373x; 1.7281x over previous
import functools

import jax
import jax.numpy as jnp
import numpy as np
from jax import lax
from jax.experimental import pallas as pl
from jax.experimental.pallas import tpu as pltpu
from jax.experimental.pallas import tpu_sc as plsc

_NC, _NS, _LANES = 2, 16, 16
_NW = _NC * _NS
_ROWS = 512
_COLS = 4096
_BLK = 64
_NB = _COLS // _BLK
_NG = _NB // _LANES
_RPW = _ROWS // _NW


def _eytzinger_perm(n=15, depth=4):
    p = np.zeros(n, dtype=np.int32)

    def build(node, lo, hi):
        if lo >= hi:
            return
        mid = (lo + hi) // 2
        p[node - 1] = mid
        build(2 * node, lo, mid)
        build(2 * node + 1, mid + 1, hi)

    build(1, 0, n)
    return p


_EY_PERM = _eytzinger_perm()

_M = (-0.853784441947937, -0.6248890161514282, -0.4795140027999878,
      -0.3638618588447571, -0.2634255886077881, -0.1716436743736267,
      -0.08471819758415222, 0.0, 0.08471819758415222, 0.1716436743736267,
      0.2634255886077881, 0.3638618588447571, 0.4795140027999878,
      0.6248890161514282, 0.853784441947937)


def _sc_body(x_hbm, ey_hbm, u_hbm,
             rec_hbm, q_hbm, sq_hbm, smn_hbm, smx_hbm, xmn_hbm,
             xrow, rrow, qrow, eyv, uv, scb, sdb, xmb, sqb, smb, sxb):
    wid = lax.axis_index("c") * _NS + lax.axis_index("s")
    base_row = wid * _RPW
    pltpu.sync_copy(ey_hbm, eyv)
    pltpu.sync_copy(u_hbm, uv)
    plsc.subcore_barrier()
    lane = lax.iota(jnp.int32, _LANES)
    lane64 = lane * _BLK

    def row_step(r, carry):
        roff = (base_row + r) * _COLS
        pltpu.sync_copy(x_hbm.at[pl.ds(roff, _COLS)], xrow)

        for g in range(_NG):
            gbase = lane64 + g * (_LANES * _BLK)

            def mm_step(e, c, gbase=gbase):
                vmn, vmx = c
                ev = (lane + e) & (_BLK - 1)
                xe = plsc.load_gather(xrow, [gbase + ev])
                return jnp.minimum(vmn, xe), jnp.maximum(vmx, xe)

            big = jnp.full((_LANES,), jnp.inf, jnp.float32)
            vmn, vmx = lax.fori_loop(0, _BLK, mm_step, (big, -big), unroll=8)
            xmb[pl.ds(r * _NB + g * _LANES, _LANES)] = vmn
            scb[pl.ds(g * _LANES, _LANES)] = vmx - vmn

        s0 = scb[pl.ds(0, _LANES)]
        s1 = scb[pl.ds(_LANES, _LANES)]
        s2 = scb[pl.ds(2 * _LANES, _LANES)]
        s3 = scb[pl.ds(3 * _LANES, _LANES)]
        smin = jnp.min(jnp.minimum(jnp.minimum(s0, s1), jnp.minimum(s2, s3)))
        smax = jnp.max(jnp.maximum(jnp.maximum(s0, s1), jnp.maximum(s2, s3)))
        dd = smax - smin + jnp.float32(1e-8)
        rng = smax - smin
        for g in range(_NG):
            sc = scb[pl.ds(g * _LANES, _LANES)]
            v = (sc - smin) / dd * jnp.float32(255.0)
            qi = (v + jnp.float32(0.5)).astype(jnp.int32)
            sqb[pl.ds(r * _NB + g * _LANES, _LANES)] = qi
            sdb[pl.ds(g * _LANES, _LANES)] = (
                smin + qi.astype(jnp.float32) / jnp.float32(255.0) * rng)
        msk = lane == r
        plsc.store_scatter(smb, [lane], jnp.full((_LANES,), smin, jnp.float32),
                           mask=msk)
        plsc.store_scatter(sxb, [lane], jnp.full((_LANES,), smax, jnp.float32),
                           mask=msk)

        for g in range(_NG):
            gbase = lane64 + g * (_LANES * _BLK)
            vmn = xmb[pl.ds(r * _NB + g * _LANES, _LANES)]
            sc = scb[pl.ds(g * _LANES, _LANES)]
            sd = sdb[pl.ds(g * _LANES, _LANES)]
            inv2 = jnp.float32(2.0) / (sc + jnp.float32(1e-8))
            xb = -vmn * inv2 - jnp.float32(1.0)
            M = _M

            def q_step(e, c, gbase=gbase, vmn=vmn, sd=sd, inv2=inv2, xb=xb):
                idx = gbase + ((lane + e) & (_BLK - 1))
                xe = plsc.load_gather(xrow, [idx])
                xn = xe * inv2 + xb
                b1 = xn > M[7]
                h2 = jnp.where(b1, M[11], M[3])
                b2 = xn > h2
                h3 = jnp.where(b1, jnp.where(b2, M[13], M[9]),
                               jnp.where(b2, M[5], M[1]))
                b3 = xn > h3
                t00 = jnp.where(b3, M[2], M[0])
                t01 = jnp.where(b3, M[6], M[4])
                t10 = jnp.where(b3, M[10], M[8])
                t11 = jnp.where(b3, M[14], M[12])
                h4 = jnp.where(b1, jnp.where(b2, t11, t10),
                               jnp.where(b2, t01, t00))
                b4 = xn > h4
                qv = (jnp.where(b1, 8, 0) + jnp.where(b2, 4, 0)
                      + jnp.where(b3, 2, 0) + jnp.where(b4, 1, 0))
                uu = plsc.load_gather(uv, [qv])
                plsc.store_scatter(rrow, [idx], uu * sd + vmn)
                plsc.store_scatter(qrow, [idx], qv)
                return c

            lax.fori_loop(0, _BLK, q_step, 0, unroll=8)

        pltpu.sync_copy(rrow, rec_hbm.at[pl.ds(roff, _COLS)])
        pltpu.sync_copy(qrow, q_hbm.at[pl.ds(roff, _COLS)])
        return carry

    lax.fori_loop(0, _RPW, row_step, 0)
    pltpu.sync_copy(xmb, xmn_hbm.at[pl.ds(base_row * _NB, _RPW * _NB)])
    pltpu.sync_copy(sqb, sq_hbm.at[pl.ds(base_row * _NB, _RPW * _NB)])
    pltpu.sync_copy(smb, smn_hbm.at[pl.ds(base_row, _RPW)])
    pltpu.sync_copy(sxb, smx_hbm.at[pl.ds(base_row, _RPW)])


@functools.cache
def _make_sc_call():
    mesh = plsc.VectorSubcoreMesh(core_axis_name="c", subcore_axis_name="s",
                                  num_cores=_NC, num_subcores=_NS)
    n = _ROWS * _COLS
    return pl.kernel(
        _sc_body,
        out_type=(
            jax.ShapeDtypeStruct((n,), jnp.float32),
            jax.ShapeDtypeStruct((n,), jnp.int32),
            jax.ShapeDtypeStruct((_ROWS * _NB,), jnp.int32),
            jax.ShapeDtypeStruct((_ROWS,), jnp.float32),
            jax.ShapeDtypeStruct((_ROWS,), jnp.float32),
            jax.ShapeDtypeStruct((_ROWS * _NB,), jnp.float32),
        ),
        mesh=mesh,
        compiler_params=pltpu.CompilerParams(needs_layout_passes=False),
        scratch_types=(
            pltpu.VMEM((_COLS,), jnp.float32),
            pltpu.VMEM((_COLS,), jnp.float32),
            pltpu.VMEM((_COLS,), jnp.int32),
            pltpu.VMEM((_LANES,), jnp.float32),
            pltpu.VMEM((_LANES,), jnp.float32),
            pltpu.VMEM((_NB,), jnp.float32),
            pltpu.VMEM((_NB,), jnp.float32),
            pltpu.VMEM((_RPW * _NB,), jnp.float32),
            pltpu.VMEM((_RPW * _NB,), jnp.int32),
            pltpu.VMEM((_RPW,), jnp.float32),
            pltpu.VMEM((_RPW,), jnp.float32),
        ),
    )


def _nf_tables():
    n_levels = 16
    p = (jnp.arange(n_levels, dtype=jnp.float32) + 0.5) / n_levels
    t = jnp.sqrt(2.0) * jax.scipy.special.erfinv(2.0 * p - 1.0)
    t = (t / jnp.max(jnp.abs(t))).astype(jnp.float32)
    u = (t + 1.0) / 2.0
    mids = (t[:-1] + t[1:]) * 0.5
    ey = jnp.concatenate([mids[_EY_PERM], jnp.zeros((1,), jnp.float32)])
    return ey.astype(jnp.float32), u.astype(jnp.float32)


def kernel(x):
    orig_shape = x.shape
    ey16, u16 = _nf_tables()
    xf = x.reshape(-1)
    rec, qf, sq, smn, smx, xmn = _make_sc_call()(xf, ey16, u16)
    return (rec.reshape(orig_shape),
            qf.reshape(_ROWS, _NB, _BLK),
            sq.astype(jnp.uint8).reshape(_ROWS, _NB),
            smn.reshape(_ROWS, 1),
            smx.reshape(_ROWS, 1),
            xmn.reshape(_ROWS, _NB, 1))


# speedup vs baseline: 3.6759x; 1.6430x over previous
import functools

import jax
import jax.numpy as jnp
import numpy as np
from jax import lax
from jax.experimental import pallas as pl
from jax.experimental.pallas import tpu as pltpu
from jax.experimental.pallas import tpu_sc as plsc

_NC, _NS, _LANES = 2, 16, 16
_NW = _NC * _NS
_ROWS = 512
_COLS = 4096
_BLK = 64
_NB = _COLS // _BLK
_NG = _NB // _LANES
_RPW = _ROWS // _NW


def _eytzinger_perm(n=15, depth=4):
    p = np.zeros(n, dtype=np.int32)

    def build(node, lo, hi):
        if lo >= hi:
            return
        mid = (lo + hi) // 2
        p[node - 1] = mid
        build(2 * node, lo, mid)
        build(2 * node + 1, mid + 1, hi)

    build(1, 0, n)
    return p


_EY_PERM = _eytzinger_perm()

_M = (-0.853784441947937, -0.6248890161514282, -0.4795140027999878,
      -0.3638618588447571, -0.2634255886077881, -0.1716436743736267,
      -0.08471819758415222, 0.0, 0.08471819758415222, 0.1716436743736267,
      0.2634255886077881, 0.3638618588447571, 0.4795140027999878,
      0.6248890161514282, 0.853784441947937)


def _sc_body(x_hbm, ey_hbm, u_hbm,
             rec_hbm, q_hbm, sq_hbm, smn_hbm, smx_hbm, xmn_hbm,
             xrow, rrow, qrow, eyv, uv, scb, sdb, xmb, sqb, smb, sxb):
    wid = lax.axis_index("c") * _NS + lax.axis_index("s")
    base_row = wid * _RPW
    pltpu.sync_copy(ey_hbm, eyv)
    pltpu.sync_copy(u_hbm, uv)
    plsc.subcore_barrier()
    lane = lax.iota(jnp.int32, _LANES)
    lane64 = lane * _BLK

    def row_step(r, carry):
        roff = (base_row + r) * _COLS
        pltpu.sync_copy(x_hbm.at[pl.ds(roff, _COLS)], xrow)

        for g in range(_NG):
            gbase = lane64 + g * (_LANES * _BLK)

            def mm_step(e, c, gbase=gbase):
                vmn, vmx = c
                ev = (lane + e) & (_BLK - 1)
                xe = plsc.load_gather(xrow, [gbase + ev])
                return jnp.minimum(vmn, xe), jnp.maximum(vmx, xe)

            big = jnp.full((_LANES,), jnp.inf, jnp.float32)
            vmn, vmx = lax.fori_loop(0, _BLK, mm_step, (big, -big), unroll=8)
            xmb[pl.ds(r * _NB + g * _LANES, _LANES)] = vmn
            scb[pl.ds(g * _LANES, _LANES)] = vmx - vmn

        s0 = scb[pl.ds(0, _LANES)]
        s1 = scb[pl.ds(_LANES, _LANES)]
        s2 = scb[pl.ds(2 * _LANES, _LANES)]
        s3 = scb[pl.ds(3 * _LANES, _LANES)]
        smin = jnp.min(jnp.minimum(jnp.minimum(s0, s1), jnp.minimum(s2, s3)))
        smax = jnp.max(jnp.maximum(jnp.maximum(s0, s1), jnp.maximum(s2, s3)))
        dd = smax - smin + jnp.float32(1e-8)
        rng = smax - smin
        for g in range(_NG):
            sc = scb[pl.ds(g * _LANES, _LANES)]
            v = (sc - smin) / dd * jnp.float32(255.0)
            qi = (v + jnp.float32(0.5)).astype(jnp.int32)
            sqb[pl.ds(r * _NB + g * _LANES, _LANES)] = qi
            sdb[pl.ds(g * _LANES, _LANES)] = (
                smin + qi.astype(jnp.float32) / jnp.float32(255.0) * rng)
        msk = lane == r
        plsc.store_scatter(smb, [lane], jnp.full((_LANES,), smin, jnp.float32),
                           mask=msk)
        plsc.store_scatter(sxb, [lane], jnp.full((_LANES,), smax, jnp.float32),
                           mask=msk)

        for g in range(_NG):
            gbase = lane64 + g * (_LANES * _BLK)
            vmn = xmb[pl.ds(r * _NB + g * _LANES, _LANES)]
            sc = scb[pl.ds(g * _LANES, _LANES)]
            sd = sdb[pl.ds(g * _LANES, _LANES)]
            inv2 = jnp.float32(2.0) / (sc + jnp.float32(1e-8))
            xb = -vmn * inv2 - jnp.float32(1.0)
            M = _M

            @plsc.parallel_loop(0, _BLK, unroll=8)
            def q_step(e, gbase=gbase, vmn=vmn, sd=sd, inv2=inv2, xb=xb):
                idx = gbase + ((lane + e) & (_BLK - 1))
                xe = plsc.load_gather(xrow, [idx])
                xn = xe * inv2 + xb
                b1 = xn > M[7]
                h2 = jnp.where(b1, M[11], M[3])
                b2 = xn > h2
                h3 = jnp.where(b1, jnp.where(b2, M[13], M[9]),
                               jnp.where(b2, M[5], M[1]))
                b3 = xn > h3
                t00 = jnp.where(b3, M[2], M[0])
                t01 = jnp.where(b3, M[6], M[4])
                t10 = jnp.where(b3, M[10], M[8])
                t11 = jnp.where(b3, M[14], M[12])
                h4 = jnp.where(b1, jnp.where(b2, t11, t10),
                               jnp.where(b2, t01, t00))
                b4 = xn > h4
                qv = (jnp.where(b1, 8, 0) + jnp.where(b2, 4, 0)
                      + jnp.where(b3, 2, 0) + jnp.where(b4, 1, 0))
                uu = plsc.load_gather(uv, [qv])
                plsc.store_scatter(rrow, [idx], uu * sd + vmn)
                plsc.store_scatter(qrow, [idx], qv)

        pltpu.sync_copy(rrow, rec_hbm.at[pl.ds(roff, _COLS)])
        pltpu.sync_copy(qrow, q_hbm.at[pl.ds(roff, _COLS)])
        return carry

    lax.fori_loop(0, _RPW, row_step, 0)
    pltpu.sync_copy(xmb, xmn_hbm.at[pl.ds(base_row * _NB, _RPW * _NB)])
    pltpu.sync_copy(sqb, sq_hbm.at[pl.ds(base_row * _NB, _RPW * _NB)])
    pltpu.sync_copy(smb, smn_hbm.at[pl.ds(base_row, _RPW)])
    pltpu.sync_copy(sxb, smx_hbm.at[pl.ds(base_row, _RPW)])


@functools.cache
def _make_sc_call():
    mesh = plsc.VectorSubcoreMesh(core_axis_name="c", subcore_axis_name="s",
                                  num_cores=_NC, num_subcores=_NS)
    n = _ROWS * _COLS
    return pl.kernel(
        _sc_body,
        out_type=(
            jax.ShapeDtypeStruct((n,), jnp.float32),
            jax.ShapeDtypeStruct((n,), jnp.int32),
            jax.ShapeDtypeStruct((_ROWS * _NB,), jnp.int32),
            jax.ShapeDtypeStruct((_ROWS,), jnp.float32),
            jax.ShapeDtypeStruct((_ROWS,), jnp.float32),
            jax.ShapeDtypeStruct((_ROWS * _NB,), jnp.float32),
        ),
        mesh=mesh,
        compiler_params=pltpu.CompilerParams(needs_layout_passes=False),
        scratch_types=(
            pltpu.VMEM((_COLS,), jnp.float32),
            pltpu.VMEM((_COLS,), jnp.float32),
            pltpu.VMEM((_COLS,), jnp.int32),
            pltpu.VMEM((_LANES,), jnp.float32),
            pltpu.VMEM((_LANES,), jnp.float32),
            pltpu.VMEM((_NB,), jnp.float32),
            pltpu.VMEM((_NB,), jnp.float32),
            pltpu.VMEM((_RPW * _NB,), jnp.float32),
            pltpu.VMEM((_RPW * _NB,), jnp.int32),
            pltpu.VMEM((_RPW,), jnp.float32),
            pltpu.VMEM((_RPW,), jnp.float32),
        ),
    )


def _nf_tables():
    n_levels = 16
    p = (jnp.arange(n_levels, dtype=jnp.float32) + 0.5) / n_levels
    t = jnp.sqrt(2.0) * jax.scipy.special.erfinv(2.0 * p - 1.0)
    t = (t / jnp.max(jnp.abs(t))).astype(jnp.float32)
    u = (t + 1.0) / 2.0
    mids = (t[:-1] + t[1:]) * 0.5
    ey = jnp.concatenate([mids[_EY_PERM], jnp.zeros((1,), jnp.float32)])
    return ey.astype(jnp.float32), u.astype(jnp.float32)


def kernel(x):
    orig_shape = x.shape
    ey16, u16 = _nf_tables()
    xf = x.reshape(-1)
    rec, qf, sq, smn, smx, xmn = _make_sc_call()(xf, ey16, u16)
    return (rec.reshape(orig_shape),
            qf.reshape(_ROWS, _NB, _BLK),
            sq.astype(jnp.uint8).reshape(_ROWS, _NB),
            smn.reshape(_ROWS, 1),
            smx.reshape(_ROWS, 1),
            xmn.reshape(_ROWS, _NB, 1))
